# baseline (speedup 1.0000x reference)
import jax
import jax.numpy as jnp
from jax import lax
from jax.experimental import pallas as pl
from jax.experimental.pallas import tpu as pltpu
from jax.experimental.pallas import tpu_sc as plsc

P, L, A = 32, 256, 28
T = 24
N_TABLES, BINS = 40, 36
NSTEP = L // 16
PL = P * L

OFF_UP = 0
OFF_PRO = OFF_UP + T
OFF_RTAB = OFF_PRO + T
OFF_DOWN = OFF_RTAB + 2 * T
OFF_TOR = OFF_DOWN + 2 * T * A
META_LEN = OFF_TOR + 8 * T


def _sc_gather_body(coords_hbm, offs_hbm, bt_hbm, irc_hbm, meta_hbm, par_hbm,
                    pts_out, tib_out,
                    c_v, offs_v, bt_v, irc_v, meta_v, par_v, obuf_v, tib_v, sem):
    cid = lax.axis_index("c")
    sid = lax.axis_index("s")
    wid = sid * 2 + cid
    copies = [
        pltpu.async_copy(coords_hbm.at[pl.ds(0, 1), pl.ds(wid * (L * A), L * A)], c_v.at[pl.ds(0, 1)], sem),
        pltpu.async_copy(coords_hbm.at[pl.ds(1, 1), pl.ds(wid * (L * A), L * A)], c_v.at[pl.ds(1, 1)], sem),
        pltpu.async_copy(coords_hbm.at[pl.ds(2, 1), pl.ds(wid * (L * A), L * A)], c_v.at[pl.ds(2, 1)], sem),
        pltpu.async_copy(offs_hbm.at[wid], offs_v, sem),
        pltpu.async_copy(bt_hbm.at[wid], bt_v, sem),
        pltpu.async_copy(irc_hbm.at[pl.ds(wid * 4, 4)], irc_v, sem),
        pltpu.async_copy(meta_hbm, meta_v, sem),
        pltpu.async_copy(par_hbm, par_v, sem),
    ]
    for c in copies:
        c.wait()

    iota = lax.iota(jnp.int32, 16)
    for s in range(NSTEP):
        sl = pl.ds(s * 16, 16)
        bt = bt_v[sl]
        off = offs_v[sl]
        up = plsc.load_gather(meta_v, [bt + OFF_UP])
        lvec = iota + s * 16
        zero = jnp.zeros((16,), jnp.int32)
        up2 = up * 2
        nb = plsc.load_gather(irc_v, [up2, lvec])
        nc = plsc.load_gather(irc_v, [up2 + 1, lvec])
        nbt = plsc.load_gather(bt_v, [nb])
        noff = plsc.load_gather(offs_v, [nb])
        down = plsc.load_gather(meta_v, [OFF_DOWN + (nbt * 2 + nc) * A])
        inter = noff + down
        ipro = plsc.load_gather(meta_v, [OFF_PRO + nbt])
        ti = plsc.load_gather(meta_v, [OFF_RTAB + bt * 2 + ipro])
        tib_v[sl] = ti * (BINS * BINS)
        p4 = ti * 4
        for k in range(4):
            obuf_v[24 + k, sl] = plsc.load_gather(par_v, [p4 + k])
        tor8 = bt * 8 + OFF_TOR
        for t in range(2):
            for j in range(4):
                ta = plsc.load_gather(meta_v, [tor8 + t * 4 + j])
                gi = jnp.where(ta >= 0, off + ta, inter)
                row = (t * 4 + j) * 3
                obuf_v[row, sl] = plsc.load_gather(c_v, [zero, gi])
                obuf_v[row + 1, sl] = plsc.load_gather(c_v, [zero + 1, gi])
                obuf_v[row + 2, sl] = plsc.load_gather(c_v, [zero + 2, gi])
    cc = [pltpu.async_copy(obuf_v, pts_out.at[:, pl.ds(wid * L, L)], sem),
          pltpu.async_copy(tib_v, tib_out.at[0, pl.ds(wid * L, L)], sem)]
    for c in cc:
        c.wait()


def _tc_transpose_body(c_ref, o_ref):
    o_ref[...] = jnp.swapaxes(c_ref[0], 0, 1)


def _dot0(t):
    return (t[0] + t[1]) + t[2]


def _dihedral_rows(pc):
    p0, p1, p2, p3 = pc[0:3], pc[3:6], pc[6:9], pc[9:12]
    b0 = [p0[i] - p1[i] for i in range(3)]
    b1 = [p2[i] - p1[i] for i in range(3)]
    b2 = [p3[i] - p2[i] for i in range(3)]
    ss = _dot0([b1[i] * b1[i] for i in range(3)])
    den = jnp.sqrt(ss) + jnp.float32(1e-8)
    b1n = [b1[i] / den for i in range(3)]
    s0 = _dot0([b0[i] * b1n[i] for i in range(3)])
    v = [b0[i] - s0 * b1n[i] for i in range(3)]
    s2 = _dot0([b2[i] * b1n[i] for i in range(3)])
    w = [b2[i] - s2 * b1n[i] for i in range(3)]
    x = _dot0([v[i] * w[i] for i in range(3)])
    cr = [b1n[(i + 1) % 3] * v[(i + 2) % 3] - b1n[(i + 2) % 3] * v[(i + 1) % 3]
          for i in range(3)]
    y = _dot0([cr[i] * w[i] for i in range(3)])
    return jnp.arctan2(y, x)


def _tc_math_body(pts_ref, tib_ref, wts_ref, gidx_ref):
    angs = []
    for t in range(2):
        pc = [pts_ref[t * 12 + r, :] for r in range(12)]
        angs.append(_dihedral_rows(pc))
    phi, psi = angs
    prm = [pts_ref[24 + k, :] for k in range(4)]
    fi = (phi - prm[0]) / prm[2]
    fj = (psi - prm[1]) / prm[3]
    i0f = jnp.floor(fi)
    j0f = jnp.floor(fj)
    a = fi - i0f
    b = fj - j0f
    i0 = jnp.mod(i0f.astype(jnp.int32), BINS)
    j0 = jnp.mod(j0f.astype(jnp.int32), BINS)
    i1 = jnp.mod(i0 + 1, BINS)
    j1 = jnp.mod(j0 + 1, BINS)
    tib = tib_ref[0, :]
    wts_ref[0, :] = (1 - a) * (1 - b)
    wts_ref[1, :] = a * (1 - b)
    wts_ref[2, :] = (1 - a) * b
    wts_ref[3, :] = a * b
    gidx_ref[0, :] = tib + i0 * BINS + j0
    gidx_ref[1, :] = tib + i1 * BINS + j0
    gidx_ref[2, :] = tib + i0 * BINS + j1
    gidx_ref[3, :] = tib + i1 * BINS + j1


def _sc_combine_body(gidx_hbm, wts_hbm, rama_hbm, out_hbm,
                     gi_v, wt_v, vals_v, out_v, sem):
    cid = lax.axis_index("c")
    sid = lax.axis_index("s")
    wid = sid * 2 + cid
    copies = []
    for k in range(4):
        copies.append(pltpu.async_copy(gidx_hbm.at[k, pl.ds(wid * L, L)],
                                       gi_v.at[pl.ds(k * L, L)], sem))
        copies.append(pltpu.async_copy(wts_hbm.at[k, pl.ds(wid * L, L)],
                                       wt_v.at[pl.ds(k * L, L)], sem))
    for c in copies:
        c.wait()
    pltpu.async_copy(rama_hbm.at[gi_v], vals_v, sem).wait()
    acc = jnp.zeros((16,), jnp.float32)
    for s in range(NSTEP):
        vals = []
        for k in range(4):
            vk = vals_v[pl.ds(k * L + s * 16, 16)]
            wk = wt_v[pl.ds(k * L + s * 16, 16)]
            vals.append(vk * wk)
        acc = acc + ((vals[0] + vals[1]) + (vals[2] + vals[3]))
    tot = jnp.sum(acc)
    out_v[...] = jnp.full((16,), tot, jnp.float32)
    pltpu.sync_copy(out_v, out_hbm.at[wid])


def kernel(coords, pose_stack_block_coord_offset, pose_stack_block_type,
           pose_stack_inter_residue_connections, bt_atom_downstream_of_conn,
           bt_rama_table, bt_upper_conn_ind, bt_is_pro, bt_rama_torsion_atoms,
           rama_tables, table_params):
    coords2 = pl.pallas_call(
        _tc_transpose_body,
        grid=(P,),
        in_specs=[pl.BlockSpec((1, L * A, 3), lambda i: (i, 0, 0))],
        out_specs=pl.BlockSpec((3, L * A), lambda i: (0, i)),
        out_shape=jax.ShapeDtypeStruct((3, P * L * A), jnp.float32),
    )(coords)
    offs2 = pose_stack_block_coord_offset.astype(jnp.int32)
    bt2 = pose_stack_block_type.astype(jnp.int32)
    irc4 = pose_stack_inter_residue_connections.astype(jnp.int32)
    irc2 = jnp.stack([irc4[:, :, 0, 0], irc4[:, :, 0, 1],
                      irc4[:, :, 1, 0], irc4[:, :, 1, 1]],
                     axis=1).reshape(P * 4, L)
    meta = jnp.concatenate([
        bt_upper_conn_ind.astype(jnp.int32),
        bt_is_pro.astype(jnp.int32),
        bt_rama_table.astype(jnp.int32).reshape(-1),
        bt_atom_downstream_of_conn.astype(jnp.int32).reshape(-1),
        bt_rama_torsion_atoms.astype(jnp.int32).reshape(-1),
    ])
    parflat = table_params.astype(jnp.float32).reshape(-1)
    ramaflat = rama_tables.astype(jnp.float32).reshape(-1)

    mesh = plsc.VectorSubcoreMesh(core_axis_name="c", subcore_axis_name="s",
                                  num_cores=2, num_subcores=16)

    gather_stage = pl.kernel(
        _sc_gather_body,
        out_type=(jax.ShapeDtypeStruct((28, PL), jnp.float32),
                  jax.ShapeDtypeStruct((1, PL), jnp.int32)),
        mesh=mesh,
        compiler_params=pltpu.CompilerParams(needs_layout_passes=False),
        scratch_types=[
            pltpu.VMEM((3, L * A), jnp.float32),
            pltpu.VMEM((L,), jnp.int32),
            pltpu.VMEM((L,), jnp.int32),
            pltpu.VMEM((4, L), jnp.int32),
            pltpu.VMEM((META_LEN,), jnp.int32),
            pltpu.VMEM((4 * N_TABLES,), jnp.float32),
            pltpu.VMEM((28, L), jnp.float32),
            pltpu.VMEM((L,), jnp.int32),
            pltpu.SemaphoreType.DMA,
        ],
    )
    pts, tib = gather_stage(coords2, offs2, bt2, irc2, meta, parflat)

    wts, gidx = pl.pallas_call(
        _tc_math_body,
        out_shape=[
            jax.ShapeDtypeStruct((4, PL), jnp.float32),
            jax.ShapeDtypeStruct((4, PL), jnp.int32),
        ],
    )(pts, tib)

    combine_stage = pl.kernel(
        _sc_combine_body,
        out_type=jax.ShapeDtypeStruct((P, 16), jnp.float32),
        mesh=mesh,
        compiler_params=pltpu.CompilerParams(needs_layout_passes=False),
        scratch_types=[
            pltpu.VMEM((4 * L,), jnp.int32),
            pltpu.VMEM((4 * L,), jnp.float32),
            pltpu.VMEM((4 * L,), jnp.float32),
            pltpu.VMEM((16,), jnp.float32),
            pltpu.SemaphoreType.DMA,
        ],
    )
    out = combine_stage(gidx, wts, ramaflat)
    return out[:, 0]


# speedup vs baseline: 1.8978x; 1.8978x over previous
import jax
import jax.numpy as jnp
from jax import lax
from jax.experimental import pallas as pl
from jax.experimental.pallas import tpu as pltpu
from jax.experimental.pallas import tpu_sc as plsc

P, L, A = 32, 256, 28
T = 24
N_TABLES, BINS = 40, 36
NSTEP = L // 16
PL = P * L

OFF_UP = 0
OFF_PRO = OFF_UP + T
OFF_RTAB = OFF_PRO + T
OFF_DOWN = OFF_RTAB + 2 * T
OFF_TOR = OFF_DOWN + 2 * T * A
META_LEN = OFF_TOR + 8 * T


def _sc_gather_body(coords_hbm, offs_hbm, bt_hbm, irc_hbm, meta_hbm, par_hbm,
                    pts_out, tib_out,
                    c_v, offs_v, bt_v, irc_v, meta_v, par_v, obuf_v, tib_v, sem):
    cid = lax.axis_index("c")
    sid = lax.axis_index("s")
    wid = sid * 2 + cid
    copies = [
        pltpu.async_copy(coords_hbm.at[wid], c_v, sem),
        pltpu.async_copy(offs_hbm.at[wid], offs_v, sem),
        pltpu.async_copy(bt_hbm.at[wid], bt_v, sem),
        pltpu.async_copy(irc_hbm.at[pl.ds(wid * 4, 4)], irc_v, sem),
        pltpu.async_copy(meta_hbm, meta_v, sem),
        pltpu.async_copy(par_hbm, par_v, sem),
    ]
    for c in copies:
        c.wait()

    iota = lax.iota(jnp.int32, 16)
    for s in range(NSTEP):
        sl = pl.ds(s * 16, 16)
        bt = bt_v[sl]
        off = offs_v[sl]
        up = plsc.load_gather(meta_v, [bt + OFF_UP])
        lvec = iota + s * 16
        zero = jnp.zeros((16,), jnp.int32)
        up2 = up * 2
        nb = plsc.load_gather(irc_v, [up2, lvec])
        nc = plsc.load_gather(irc_v, [up2 + 1, lvec])
        nbt = plsc.load_gather(bt_v, [nb])
        noff = plsc.load_gather(offs_v, [nb])
        down = plsc.load_gather(meta_v, [OFF_DOWN + (nbt * 2 + nc) * A])
        inter = noff + down
        ipro = plsc.load_gather(meta_v, [OFF_PRO + nbt])
        ti = plsc.load_gather(meta_v, [OFF_RTAB + bt * 2 + ipro])
        tib_v[sl] = ti * (BINS * BINS)
        p4 = ti * 4
        for k in range(4):
            obuf_v[24 + k, sl] = plsc.load_gather(par_v, [p4 + k])
        tor8 = bt * 8 + OFF_TOR
        for t in range(2):
            for j in range(4):
                ta = plsc.load_gather(meta_v, [tor8 + t * 4 + j])
                gi = jnp.where(ta >= 0, off + ta, inter)
                g3 = gi * 3
                row = (t * 4 + j) * 3
                obuf_v[row, sl] = plsc.load_gather(c_v, [g3])
                obuf_v[row + 1, sl] = plsc.load_gather(c_v, [g3 + 1])
                obuf_v[row + 2, sl] = plsc.load_gather(c_v, [g3 + 2])
    cc = [pltpu.async_copy(obuf_v, pts_out.at[:, pl.ds(wid * L, L)], sem),
          pltpu.async_copy(tib_v, tib_out.at[0, pl.ds(wid * L, L)], sem)]
    for c in cc:
        c.wait()


def _dot0(t):
    return (t[0] + t[1]) + t[2]


def _dihedral_rows(pc):
    p0, p1, p2, p3 = pc[0:3], pc[3:6], pc[6:9], pc[9:12]
    b0 = [p0[i] - p1[i] for i in range(3)]
    b1 = [p2[i] - p1[i] for i in range(3)]
    b2 = [p3[i] - p2[i] for i in range(3)]
    ss = _dot0([b1[i] * b1[i] for i in range(3)])
    den = jnp.sqrt(ss) + jnp.float32(1e-8)
    b1n = [b1[i] / den for i in range(3)]
    s0 = _dot0([b0[i] * b1n[i] for i in range(3)])
    v = [b0[i] - s0 * b1n[i] for i in range(3)]
    s2 = _dot0([b2[i] * b1n[i] for i in range(3)])
    w = [b2[i] - s2 * b1n[i] for i in range(3)]
    x = _dot0([v[i] * w[i] for i in range(3)])
    cr = [b1n[(i + 1) % 3] * v[(i + 2) % 3] - b1n[(i + 2) % 3] * v[(i + 1) % 3]
          for i in range(3)]
    y = _dot0([cr[i] * w[i] for i in range(3)])
    return jnp.arctan2(y, x)


def _tc_math_body(pts_ref, tib_ref, wts_ref, gidx_ref):
    angs = []
    for t in range(2):
        pc = [pts_ref[t * 12 + r, :] for r in range(12)]
        angs.append(_dihedral_rows(pc))
    phi, psi = angs
    prm = [pts_ref[24 + k, :] for k in range(4)]
    fi = (phi - prm[0]) / prm[2]
    fj = (psi - prm[1]) / prm[3]
    i0f = jnp.floor(fi)
    j0f = jnp.floor(fj)
    a = fi - i0f
    b = fj - j0f
    i0 = jnp.mod(i0f.astype(jnp.int32), BINS)
    j0 = jnp.mod(j0f.astype(jnp.int32), BINS)
    i1 = jnp.mod(i0 + 1, BINS)
    j1 = jnp.mod(j0 + 1, BINS)
    tib = tib_ref[0, :]
    wts_ref[0, :] = (1 - a) * (1 - b)
    wts_ref[1, :] = a * (1 - b)
    wts_ref[2, :] = (1 - a) * b
    wts_ref[3, :] = a * b
    gidx_ref[0, :] = tib + i0 * BINS + j0
    gidx_ref[1, :] = tib + i1 * BINS + j0
    gidx_ref[2, :] = tib + i0 * BINS + j1
    gidx_ref[3, :] = tib + i1 * BINS + j1


def _sc_combine_body(gidx_hbm, wts_hbm, rama_hbm, out_hbm,
                     gi_v, wt_v, vals_v, out_v, sem):
    cid = lax.axis_index("c")
    sid = lax.axis_index("s")
    wid = sid * 2 + cid
    copies = []
    for k in range(4):
        copies.append(pltpu.async_copy(gidx_hbm.at[k, pl.ds(wid * L, L)],
                                       gi_v.at[pl.ds(k * L, L)], sem))
        copies.append(pltpu.async_copy(wts_hbm.at[k, pl.ds(wid * L, L)],
                                       wt_v.at[pl.ds(k * L, L)], sem))
    for c in copies:
        c.wait()
    pltpu.async_copy(rama_hbm.at[gi_v], vals_v, sem).wait()
    acc = jnp.zeros((16,), jnp.float32)
    for s in range(NSTEP):
        vals = []
        for k in range(4):
            vk = vals_v[pl.ds(k * L + s * 16, 16)]
            wk = wt_v[pl.ds(k * L + s * 16, 16)]
            vals.append(vk * wk)
        acc = acc + ((vals[0] + vals[1]) + (vals[2] + vals[3]))
    tot = jnp.sum(acc)
    out_v[...] = jnp.full((16,), tot, jnp.float32)
    pltpu.sync_copy(out_v, out_hbm.at[wid])


def kernel(coords, pose_stack_block_coord_offset, pose_stack_block_type,
           pose_stack_inter_residue_connections, bt_atom_downstream_of_conn,
           bt_rama_table, bt_upper_conn_ind, bt_is_pro, bt_rama_torsion_atoms,
           rama_tables, table_params):
    coords2 = coords.reshape(P, L * A * 3)
    offs2 = pose_stack_block_coord_offset.astype(jnp.int32)
    bt2 = pose_stack_block_type.astype(jnp.int32)
    irc4 = pose_stack_inter_residue_connections.astype(jnp.int32)
    irc2 = jnp.stack([irc4[:, :, 0, 0], irc4[:, :, 0, 1],
                      irc4[:, :, 1, 0], irc4[:, :, 1, 1]],
                     axis=1).reshape(P * 4, L)
    meta = jnp.concatenate([
        bt_upper_conn_ind.astype(jnp.int32),
        bt_is_pro.astype(jnp.int32),
        bt_rama_table.astype(jnp.int32).reshape(-1),
        bt_atom_downstream_of_conn.astype(jnp.int32).reshape(-1),
        bt_rama_torsion_atoms.astype(jnp.int32).reshape(-1),
    ])
    parflat = table_params.astype(jnp.float32).reshape(-1)
    ramaflat = rama_tables.astype(jnp.float32).reshape(-1)

    mesh = plsc.VectorSubcoreMesh(core_axis_name="c", subcore_axis_name="s",
                                  num_cores=2, num_subcores=16)

    gather_stage = pl.kernel(
        _sc_gather_body,
        out_type=(jax.ShapeDtypeStruct((28, PL), jnp.float32),
                  jax.ShapeDtypeStruct((1, PL), jnp.int32)),
        mesh=mesh,
        compiler_params=pltpu.CompilerParams(needs_layout_passes=False),
        scratch_types=[
            pltpu.VMEM((L * A * 3,), jnp.float32),
            pltpu.VMEM((L,), jnp.int32),
            pltpu.VMEM((L,), jnp.int32),
            pltpu.VMEM((4, L), jnp.int32),
            pltpu.VMEM((META_LEN,), jnp.int32),
            pltpu.VMEM((4 * N_TABLES,), jnp.float32),
            pltpu.VMEM((28, L), jnp.float32),
            pltpu.VMEM((L,), jnp.int32),
            pltpu.SemaphoreType.DMA,
        ],
    )
    pts, tib = gather_stage(coords2, offs2, bt2, irc2, meta, parflat)

    wts, gidx = pl.pallas_call(
        _tc_math_body,
        out_shape=[
            jax.ShapeDtypeStruct((4, PL), jnp.float32),
            jax.ShapeDtypeStruct((4, PL), jnp.int32),
        ],
    )(pts, tib)

    combine_stage = pl.kernel(
        _sc_combine_body,
        out_type=jax.ShapeDtypeStruct((P, 16), jnp.float32),
        mesh=mesh,
        compiler_params=pltpu.CompilerParams(needs_layout_passes=False),
        scratch_types=[
            pltpu.VMEM((4 * L,), jnp.int32),
            pltpu.VMEM((4 * L,), jnp.float32),
            pltpu.VMEM((4 * L,), jnp.float32),
            pltpu.VMEM((16,), jnp.float32),
            pltpu.SemaphoreType.DMA,
        ],
    )
    out = combine_stage(gidx, wts, ramaflat)
    return out[:, 0]


# speedup vs baseline: 3.2986x; 1.7381x over previous
import jax
import jax.numpy as jnp
from jax import lax
from jax.experimental import pallas as pl
from jax.experimental.pallas import tpu as pltpu
from jax.experimental.pallas import tpu_sc as plsc

P, L, A = 32, 256, 28
T = 24
N_TABLES, BINS = 40, 36
NSTEP = L // 16
PL = P * L

OFF_UP = 0
OFF_PRO = OFF_UP + T
OFF_RTAB = OFF_PRO + T
OFF_DOWN = OFF_RTAB + 2 * T
OFF_TOR = OFF_DOWN + 2 * T * A
META_LEN = OFF_TOR + 8 * T


def _sc_gather_body(coords_hbm, offs_hbm, bt_hbm, irc_hbm, meta_hbm, par_hbm,
                    pts_out,
                    c_v, offs_v, bt_v, irc_v, meta_v, par_v, obuf_v, sem):
    cid = lax.axis_index("c")
    sid = lax.axis_index("s")
    wid = sid * 2 + cid
    copies = [
        pltpu.async_copy(coords_hbm.at[pl.ds(0, 1), pl.ds(wid * (L * A), L * A)],
                         c_v.at[pl.ds(0, 1)], sem),
        pltpu.async_copy(coords_hbm.at[pl.ds(1, 1), pl.ds(wid * (L * A), L * A)],
                         c_v.at[pl.ds(1, 1)], sem),
        pltpu.async_copy(coords_hbm.at[pl.ds(2, 1), pl.ds(wid * (L * A), L * A)],
                         c_v.at[pl.ds(2, 1)], sem),
        pltpu.async_copy(offs_hbm.at[wid], offs_v, sem),
        pltpu.async_copy(bt_hbm.at[wid], bt_v, sem),
        pltpu.async_copy(irc_hbm.at[pl.ds(wid * 4, 4)], irc_v, sem),
        pltpu.async_copy(meta_hbm, meta_v, sem),
        pltpu.async_copy(par_hbm, par_v, sem),
    ]
    for c in copies:
        c.wait()

    iota = lax.iota(jnp.int32, 16)
    for s in range(NSTEP):
        sl = pl.ds(s * 16, 16)
        bt = bt_v[sl]
        off = offs_v[sl]
        up = plsc.load_gather(meta_v, [bt + OFF_UP])
        lvec = iota + s * 16
        zero = jnp.zeros((16,), jnp.int32)
        up2 = up * 2
        nb = plsc.load_gather(irc_v, [up2, lvec])
        nc = plsc.load_gather(irc_v, [up2 + 1, lvec])
        nbt = plsc.load_gather(bt_v, [nb])
        noff = plsc.load_gather(offs_v, [nb])
        down = plsc.load_gather(meta_v, [OFF_DOWN + (nbt * 2 + nc) * A])
        inter = noff + down
        ipro = plsc.load_gather(meta_v, [OFF_PRO + nbt])
        ti = plsc.load_gather(meta_v, [OFF_RTAB + bt * 2 + ipro])
        obuf_v[28, sl] = ti.astype(jnp.float32)
        p4 = ti * 4
        for k in range(4):
            obuf_v[24 + k, sl] = plsc.load_gather(par_v, [p4 + k])
        tor8 = bt * 8 + OFF_TOR
        for t in range(2):
            for j in range(4):
                ta = plsc.load_gather(meta_v, [tor8 + t * 4 + j])
                gi = jnp.where(ta >= 0, off + ta, inter)
                row = (t * 4 + j) * 3
                obuf_v[row, sl] = plsc.load_gather(c_v, [zero, gi])
                obuf_v[row + 1, sl] = plsc.load_gather(c_v, [zero + 1, gi])
                obuf_v[row + 2, sl] = plsc.load_gather(c_v, [zero + 2, gi])
    pltpu.sync_copy(obuf_v, pts_out.at[:, pl.ds(wid * L, L)])


def _dot0(t):
    return (t[0] + t[1]) + t[2]


def _dihedral_rows(pc):
    p0, p1, p2, p3 = pc[0:3], pc[3:6], pc[6:9], pc[9:12]
    b0 = [p0[i] - p1[i] for i in range(3)]
    b1 = [p2[i] - p1[i] for i in range(3)]
    b2 = [p3[i] - p2[i] for i in range(3)]
    ss = _dot0([b1[i] * b1[i] for i in range(3)])
    den = jnp.sqrt(ss) + jnp.float32(1e-8)
    b1n = [b1[i] / den for i in range(3)]
    s0 = _dot0([b0[i] * b1n[i] for i in range(3)])
    v = [b0[i] - s0 * b1n[i] for i in range(3)]
    s2 = _dot0([b2[i] * b1n[i] for i in range(3)])
    w = [b2[i] - s2 * b1n[i] for i in range(3)]
    x = _dot0([v[i] * w[i] for i in range(3)])
    cr = [b1n[(i + 1) % 3] * v[(i + 2) % 3] - b1n[(i + 2) % 3] * v[(i + 1) % 3]
          for i in range(3)]
    y = _dot0([cr[i] * w[i] for i in range(3)])
    return jnp.arctan2(y, x)


def _tc_math_body(pts_ref, wts_ref, gidx_ref):
    angs = []
    for t in range(2):
        pc = [pts_ref[t * 12 + r, :] for r in range(12)]
        angs.append(_dihedral_rows(pc))
    phi, psi = angs
    prm = [pts_ref[24 + k, :] for k in range(4)]
    fi = (phi - prm[0]) / prm[2]
    fj = (psi - prm[1]) / prm[3]
    i0f = jnp.floor(fi)
    j0f = jnp.floor(fj)
    a = fi - i0f
    b = fj - j0f
    i0 = jnp.mod(i0f.astype(jnp.int32), BINS)
    j0 = jnp.mod(j0f.astype(jnp.int32), BINS)
    i1 = jnp.mod(i0 + 1, BINS)
    j1 = jnp.mod(j0 + 1, BINS)
    tib = pts_ref[28, :].astype(jnp.int32) * (BINS * BINS)
    wts_ref[0, :] = (1 - a) * (1 - b)
    wts_ref[1, :] = a * (1 - b)
    wts_ref[2, :] = (1 - a) * b
    wts_ref[3, :] = a * b
    gidx_ref[0, :] = tib + i0 * BINS + j0
    gidx_ref[1, :] = tib + i1 * BINS + j0
    gidx_ref[2, :] = tib + i0 * BINS + j1
    gidx_ref[3, :] = tib + i1 * BINS + j1


def _sc_combine_body(gidx_hbm, wts_hbm, rama_hbm, out_hbm,
                     gi_v, wt_v, vals_v, out_v, sem):
    cid = lax.axis_index("c")
    sid = lax.axis_index("s")
    wid = sid * 2 + cid
    copies = []
    for k in range(4):
        copies.append(pltpu.async_copy(gidx_hbm.at[k, pl.ds(wid * L, L)],
                                       gi_v.at[pl.ds(k * L, L)], sem))
        copies.append(pltpu.async_copy(wts_hbm.at[k, pl.ds(wid * L, L)],
                                       wt_v.at[pl.ds(k * L, L)], sem))
    for c in copies:
        c.wait()
    pltpu.async_copy(rama_hbm.at[gi_v], vals_v, sem).wait()
    acc = jnp.zeros((16,), jnp.float32)
    for s in range(NSTEP):
        vals = []
        for k in range(4):
            vk = vals_v[pl.ds(k * L + s * 16, 16)]
            wk = wt_v[pl.ds(k * L + s * 16, 16)]
            vals.append(vk * wk)
        acc = acc + ((vals[0] + vals[1]) + (vals[2] + vals[3]))
    tot = jnp.sum(acc)
    out_v[...] = jnp.full((16,), tot, jnp.float32)
    pltpu.sync_copy(out_v, out_hbm.at[wid])


def kernel(coords, pose_stack_block_coord_offset, pose_stack_block_type,
           pose_stack_inter_residue_connections, bt_atom_downstream_of_conn,
           bt_rama_table, bt_upper_conn_ind, bt_is_pro, bt_rama_torsion_atoms,
           rama_tables, table_params):
    coords2 = jnp.transpose(coords, (2, 0, 1)).reshape(3, P * L * A)
    offs2 = pose_stack_block_coord_offset.astype(jnp.int32)
    bt2 = pose_stack_block_type.astype(jnp.int32)
    irc2 = jnp.transpose(pose_stack_inter_residue_connections.astype(jnp.int32),
                         (0, 2, 3, 1)).reshape(P * 4, L)
    meta = jnp.concatenate([
        bt_upper_conn_ind.astype(jnp.int32),
        bt_is_pro.astype(jnp.int32),
        bt_rama_table.astype(jnp.int32).reshape(-1),
        bt_atom_downstream_of_conn.astype(jnp.int32).reshape(-1),
        bt_rama_torsion_atoms.astype(jnp.int32).reshape(-1),
    ])
    parflat = table_params.astype(jnp.float32).reshape(-1)
    ramaflat = rama_tables.astype(jnp.float32).reshape(-1)

    mesh = plsc.VectorSubcoreMesh(core_axis_name="c", subcore_axis_name="s",
                                  num_cores=2, num_subcores=16)

    gather_stage = pl.kernel(
        _sc_gather_body,
        out_type=jax.ShapeDtypeStruct((32, PL), jnp.float32),
        mesh=mesh,
        compiler_params=pltpu.CompilerParams(needs_layout_passes=False),
        scratch_types=[
            pltpu.VMEM((3, L * A), jnp.float32),
            pltpu.VMEM((L,), jnp.int32),
            pltpu.VMEM((L,), jnp.int32),
            pltpu.VMEM((4, L), jnp.int32),
            pltpu.VMEM((META_LEN,), jnp.int32),
            pltpu.VMEM((4 * N_TABLES,), jnp.float32),
            pltpu.VMEM((32, L), jnp.float32),
            pltpu.SemaphoreType.DMA,
        ],
    )
    pts = gather_stage(coords2, offs2, bt2, irc2, meta, parflat)

    wts, gidx = pl.pallas_call(
        _tc_math_body,
        out_shape=[
            jax.ShapeDtypeStruct((8, PL), jnp.float32),
            jax.ShapeDtypeStruct((8, PL), jnp.int32),
        ],
    )(pts)

    combine_stage = pl.kernel(
        _sc_combine_body,
        out_type=jax.ShapeDtypeStruct((P, 16), jnp.float32),
        mesh=mesh,
        compiler_params=pltpu.CompilerParams(needs_layout_passes=False),
        scratch_types=[
            pltpu.VMEM((4 * L,), jnp.int32),
            pltpu.VMEM((4 * L,), jnp.float32),
            pltpu.VMEM((4 * L,), jnp.float32),
            pltpu.VMEM((16,), jnp.float32),
            pltpu.SemaphoreType.DMA,
        ],
    )
    out = combine_stage(gidx, wts, ramaflat)
    return out[:, 0]


# speedup vs baseline: 3.3242x; 1.0078x over previous
import jax
import jax.numpy as jnp
from jax import lax
from jax.experimental import pallas as pl
from jax.experimental.pallas import tpu as pltpu
from jax.experimental.pallas import tpu_sc as plsc

P, L, A = 32, 256, 28
T = 24
N_TABLES, BINS = 40, 36
NSTEP = L // 16
PL = P * L

OFF_UP = 0
OFF_PRO = OFF_UP + T
OFF_RTAB = OFF_PRO + T
OFF_DOWN = OFF_RTAB + 2 * T
OFF_TOR = OFF_DOWN + 2 * T * A
META_LEN = OFF_TOR + 8 * T


def _sc_gather_body(coords_hbm, offs_hbm, bt_hbm, irc_hbm, meta_hbm, par_hbm,
                    pts_out,
                    c_v, offs_v, bt_v, irc_v, meta_v, par_v, obuf_v, sem):
    cid = lax.axis_index("c")
    sid = lax.axis_index("s")
    wid = sid * 2 + cid
    copies = [
        pltpu.async_copy(coords_hbm.at[pl.ds(0, 1), pl.ds(wid, 1)],
                         c_v.at[pl.ds(0, 1)], sem),
        pltpu.async_copy(coords_hbm.at[pl.ds(1, 1), pl.ds(wid, 1)],
                         c_v.at[pl.ds(1, 1)], sem),
        pltpu.async_copy(coords_hbm.at[pl.ds(2, 1), pl.ds(wid, 1)],
                         c_v.at[pl.ds(2, 1)], sem),
        pltpu.async_copy(offs_hbm.at[wid], offs_v, sem),
        pltpu.async_copy(bt_hbm.at[wid], bt_v, sem),
        pltpu.async_copy(irc_hbm.at[pl.ds(wid * 4, 4)], irc_v, sem),
        pltpu.async_copy(meta_hbm, meta_v, sem),
        pltpu.async_copy(par_hbm, par_v, sem),
    ]
    for c in copies:
        c.wait()

    iota = lax.iota(jnp.int32, 16)
    for s in range(NSTEP):
        sl = pl.ds(s * 16, 16)
        bt = bt_v[sl]
        off = offs_v[sl]
        up = plsc.load_gather(meta_v, [bt + OFF_UP])
        lvec = iota + s * 16
        zero = jnp.zeros((16,), jnp.int32)
        up2 = up * 2
        nb = plsc.load_gather(irc_v, [up2, lvec])
        nc = plsc.load_gather(irc_v, [up2 + 1, lvec])
        nbt = plsc.load_gather(bt_v, [nb])
        noff = plsc.load_gather(offs_v, [nb])
        down = plsc.load_gather(meta_v, [OFF_DOWN + (nbt * 2 + nc) * A])
        inter = noff + down
        ipro = plsc.load_gather(meta_v, [OFF_PRO + nbt])
        ti = plsc.load_gather(meta_v, [OFF_RTAB + bt * 2 + ipro])
        obuf_v[28, sl] = ti.astype(jnp.float32)
        p4 = ti * 4
        for k in range(4):
            obuf_v[24 + k, sl] = plsc.load_gather(par_v, [p4 + k])
        tor8 = bt * 8 + OFF_TOR
        for t in range(2):
            for j in range(4):
                ta = plsc.load_gather(meta_v, [tor8 + t * 4 + j])
                gi = jnp.where(ta >= 0, off + ta, inter)
                row = (t * 4 + j) * 3
                obuf_v[row, sl] = plsc.load_gather(c_v, [zero, zero, gi])
                obuf_v[row + 1, sl] = plsc.load_gather(c_v, [zero + 1, zero, gi])
                obuf_v[row + 2, sl] = plsc.load_gather(c_v, [zero + 2, zero, gi])
    pltpu.sync_copy(obuf_v, pts_out.at[:, pl.ds(wid * L, L)])


def _dot0(t):
    return (t[0] + t[1]) + t[2]


def _dihedral_rows(pc):
    p0, p1, p2, p3 = pc[0:3], pc[3:6], pc[6:9], pc[9:12]
    b0 = [p0[i] - p1[i] for i in range(3)]
    b1 = [p2[i] - p1[i] for i in range(3)]
    b2 = [p3[i] - p2[i] for i in range(3)]
    ss = _dot0([b1[i] * b1[i] for i in range(3)])
    den = jnp.sqrt(ss) + jnp.float32(1e-8)
    b1n = [b1[i] / den for i in range(3)]
    s0 = _dot0([b0[i] * b1n[i] for i in range(3)])
    v = [b0[i] - s0 * b1n[i] for i in range(3)]
    s2 = _dot0([b2[i] * b1n[i] for i in range(3)])
    w = [b2[i] - s2 * b1n[i] for i in range(3)]
    x = _dot0([v[i] * w[i] for i in range(3)])
    cr = [b1n[(i + 1) % 3] * v[(i + 2) % 3] - b1n[(i + 2) % 3] * v[(i + 1) % 3]
          for i in range(3)]
    y = _dot0([cr[i] * w[i] for i in range(3)])
    return jnp.arctan2(y, x)


def _tc_math_body(pts_ref, wts_ref, gidx_ref):
    angs = []
    for t in range(2):
        pc = [pts_ref[t * 12 + r, :] for r in range(12)]
        angs.append(_dihedral_rows(pc))
    phi, psi = angs
    prm = [pts_ref[24 + k, :] for k in range(4)]
    fi = (phi - prm[0]) / prm[2]
    fj = (psi - prm[1]) / prm[3]
    i0f = jnp.floor(fi)
    j0f = jnp.floor(fj)
    a = fi - i0f
    b = fj - j0f
    i0 = jnp.mod(i0f.astype(jnp.int32), BINS)
    j0 = jnp.mod(j0f.astype(jnp.int32), BINS)
    i1 = jnp.mod(i0 + 1, BINS)
    j1 = jnp.mod(j0 + 1, BINS)
    tib = pts_ref[28, :].astype(jnp.int32) * (BINS * BINS)
    wts_ref[0, :] = (1 - a) * (1 - b)
    wts_ref[1, :] = a * (1 - b)
    wts_ref[2, :] = (1 - a) * b
    wts_ref[3, :] = a * b
    gidx_ref[0, :] = tib + i0 * BINS + j0
    gidx_ref[1, :] = tib + i1 * BINS + j0
    gidx_ref[2, :] = tib + i0 * BINS + j1
    gidx_ref[3, :] = tib + i1 * BINS + j1


def _sc_combine_body(gidx_hbm, wts_hbm, rama_hbm, out_hbm,
                     gi_v, wt_v, vals_v, out_v, sem):
    cid = lax.axis_index("c")
    sid = lax.axis_index("s")
    wid = sid * 2 + cid
    copies = [pltpu.async_copy(wts_hbm.at[:, pl.ds(wid * L, L)], wt_v, sem)]
    for k in range(4):
        copies.append(pltpu.async_copy(gidx_hbm.at[k, pl.ds(wid * L, L)],
                                       gi_v.at[pl.ds(k * L, L)], sem))
    for c in copies:
        c.wait()
    pltpu.async_copy(rama_hbm.at[gi_v], vals_v, sem).wait()
    acc = jnp.zeros((16,), jnp.float32)
    for s in range(NSTEP):
        sl = pl.ds(s * 16, 16)
        vals = [vals_v[pl.ds(k * L + s * 16, 16)] * wt_v[k, sl] for k in range(4)]
        acc = acc + ((vals[0] + vals[1]) + (vals[2] + vals[3]))
    tot = jnp.sum(acc)
    out_v[...] = jnp.full((16,), tot, jnp.float32)
    pltpu.sync_copy(out_v, out_hbm.at[wid])


def kernel(coords, pose_stack_block_coord_offset, pose_stack_block_type,
           pose_stack_inter_residue_connections, bt_atom_downstream_of_conn,
           bt_rama_table, bt_upper_conn_ind, bt_is_pro, bt_rama_torsion_atoms,
           rama_tables, table_params):
    coords2 = jnp.transpose(coords, (2, 0, 1))
    offs2 = pose_stack_block_coord_offset.astype(jnp.int32)
    bt2 = pose_stack_block_type.astype(jnp.int32)
    irc2 = jnp.transpose(pose_stack_inter_residue_connections.astype(jnp.int32),
                         (0, 2, 3, 1)).reshape(P * 4, L)
    meta = jnp.concatenate([
        bt_upper_conn_ind.astype(jnp.int32),
        bt_is_pro.astype(jnp.int32),
        bt_rama_table.astype(jnp.int32).reshape(-1),
        bt_atom_downstream_of_conn.astype(jnp.int32).reshape(-1),
        bt_rama_torsion_atoms.astype(jnp.int32).reshape(-1),
    ])
    parflat = table_params.astype(jnp.float32).reshape(-1)
    ramaflat = rama_tables.astype(jnp.float32).reshape(-1)

    mesh = plsc.VectorSubcoreMesh(core_axis_name="c", subcore_axis_name="s",
                                  num_cores=2, num_subcores=16)

    gather_stage = pl.kernel(
        _sc_gather_body,
        out_type=jax.ShapeDtypeStruct((32, PL), jnp.float32),
        mesh=mesh,
        compiler_params=pltpu.CompilerParams(needs_layout_passes=False),
        scratch_types=[
            pltpu.VMEM((3, 1, L * A), jnp.float32),
            pltpu.VMEM((L,), jnp.int32),
            pltpu.VMEM((L,), jnp.int32),
            pltpu.VMEM((4, L), jnp.int32),
            pltpu.VMEM((META_LEN,), jnp.int32),
            pltpu.VMEM((4 * N_TABLES,), jnp.float32),
            pltpu.VMEM((32, L), jnp.float32),
            pltpu.SemaphoreType.DMA,
        ],
    )
    pts = gather_stage(coords2, offs2, bt2, irc2, meta, parflat)

    wts, gidx = pl.pallas_call(
        _tc_math_body,
        out_shape=[
            jax.ShapeDtypeStruct((8, PL), jnp.float32),
            jax.ShapeDtypeStruct((8, PL), jnp.int32),
        ],
    )(pts)

    combine_stage = pl.kernel(
        _sc_combine_body,
        out_type=jax.ShapeDtypeStruct((P, 16), jnp.float32),
        mesh=mesh,
        compiler_params=pltpu.CompilerParams(needs_layout_passes=False),
        scratch_types=[
            pltpu.VMEM((4 * L,), jnp.int32),
            pltpu.VMEM((8, L), jnp.float32),
            pltpu.VMEM((4 * L,), jnp.float32),
            pltpu.VMEM((16,), jnp.float32),
            pltpu.SemaphoreType.DMA,
        ],
    )
    out = combine_stage(gidx, wts, ramaflat)
    return out[:, 0]
